# baseline (speedup 1.0000x reference)
import functools

import jax
import jax.numpy as jnp
from jax import lax
from jax.experimental import pallas as pl
from jax.experimental.pallas import tpu as pltpu
from jax.experimental.pallas import tpu_sc as plsc

F = 26
DH = 16
DM = 64
NC = 2
NS = 16
NW = NC * NS
CHUNK = 128


def _prep(xt, off2):
    B = xt.shape[1]
    Bb = 4096

    def body(x_ref, off_ref, out_ref):
        out_ref[...] = x_ref[...] + off_ref[...]

    return pl.pallas_call(
        body,
        grid=(B // Bb,),
        in_specs=[
            pl.BlockSpec((F, Bb), lambda j: (0, j)),
            pl.BlockSpec((F, 1), lambda j: (0, 0)),
        ],
        out_specs=pl.BlockSpec((F, Bb), lambda j: (0, j)),
        out_shape=jax.ShapeDtypeStruct((F, B), jnp.int32),
    )(xt, off2)


def _tpose(tT):
    V = tT.shape[1]
    CB = 16000

    def body(in_ref, out_ref):
        v = in_ref[...]
        vt = v.T.reshape(CB // 8, 8, DH)
        out_ref[...] = jnp.concatenate(
            [vt[:, e, :] for e in range(8)], axis=1
        )

    return pl.pallas_call(
        body,
        grid=(V // CB,),
        in_specs=[pl.BlockSpec((DH, CB), lambda j: (0, j))],
        out_specs=pl.BlockSpec((CB // 8, 128), lambda j: (j, 0)),
        out_shape=jax.ShapeDtypeStruct((V // 8, 128), jnp.float32),
    )(tT)


def _gather(xtg, table):
    B = xtg.shape[1]
    bpw = B // NW
    nch = bpw // CHUNK
    mesh = plsc.VectorSubcoreMesh(core_axis_name="c", subcore_axis_name="s")

    @functools.partial(
        pl.kernel,
        mesh=mesh,
        compiler_params=pltpu.CompilerParams(use_tc_tiling_on_sc=False),
        out_type=jax.ShapeDtypeStruct((F, B, DH), jnp.float32),
        scratch_types=[
            pltpu.VMEM((nch, CHUNK), jnp.int32),
            pltpu.VMEM((bpw, DH), jnp.float32),
            pltpu.SemaphoreType.DMA,
        ],
    )
    def k(xtg_hbm, table_hbm, emb_hbm, idx_v, rows_v, sem):
        wid = lax.axis_index("s") * NC + lax.axis_index("c")
        base = wid * bpw

        def body(f, _):
            for c in range(nch):
                pltpu.sync_copy(
                    xtg_hbm.at[f, pl.ds(base + c * CHUNK, CHUNK)],
                    idx_v.at[c],
                )
            copies = []
            for c in range(nch):
                copies.append(
                    pltpu.async_copy(
                        table_hbm.at[idx_v.at[c]],
                        rows_v.at[pl.ds(c * CHUNK, CHUNK)],
                        sem,
                    )
                )
            for cp in copies:
                cp.wait()
            pltpu.sync_copy(rows_v, emb_hbm.at[f, pl.ds(base, bpw)])
            return ()

        lax.fori_loop(0, F, body, (), unroll=False)

    return k(xtg, table)


def _mm2(emb3, bias, W):
    B = emb3.shape[1] * 8
    Bb = 2048

    def body(emb_ref, bias_ref, w_ref, out_ref):
        f = pl.program_id(0)
        e8 = emb_ref[0]
        h = jnp.concatenate(
            [e8[:, e * DH:(e + 1) * DH][:, None, :] for e in range(8)], axis=1
        ).reshape(Bb, DH)
        h = h + bias_ref[pl.ds(f, 1), :]
        out_ref[0] = jax.lax.dot_general(
            w_ref[...],
            h,
            (((0,), (1,)), ((), ())),
            preferred_element_type=jnp.float32,
        )

    return pl.pallas_call(
        body,
        grid=(F, B // Bb),
        in_specs=[
            pl.BlockSpec((1, Bb // 8, 128), lambda f, j: (f, j, 0)),
            pl.BlockSpec((F, DH), lambda f, j: (0, 0)),
            pl.BlockSpec((DH, DM), lambda f, j: (0, 0)),
        ],
        out_specs=pl.BlockSpec((1, DM, Bb), lambda f, j: (f, 0, j)),
        out_shape=jax.ShapeDtypeStruct((F, DM, B), jnp.float32),
    )(emb3, bias, W)


def kernel(x, table, bias, W, offsets):
    B = x.shape[0]
    xt = jnp.transpose(x).astype(jnp.int32)
    tT = jnp.transpose(table)
    xtg = _prep(xt, offsets.astype(jnp.int32)[:, None])
    t8 = _tpose(tT)
    tlin = t8.reshape(-1).reshape(table.shape[0], DH)
    emb = _gather(xtg, tlin)
    emb3 = emb.reshape(F, B // 8, 128)
    out3 = _mm2(emb3, bias, W)
    return jnp.transpose(out3, (2, 0, 1))
